# baseline (speedup 1.0000x reference)
import functools

import jax
import jax.numpy as jnp
from jax import lax
from jax.experimental import pallas as pl
from jax.experimental.pallas import tpu as pltpu
from jax.experimental.pallas import tpu_sc as plsc

N, E, D, H, Z, G = 10000, 320000, 128, 64, 32, 64

NC, NS = 2, 16
NW = NC * NS
TW = H + 16
EW = E // NW
CH = 80
NCHUNK = EW // CH
NBUF = 5
NOUTER = NCHUNK // NBUF
ZR = N // NS


def _table_body(x_ref, w1_ref, wdecpt_ref, wmut_ref, bdecp_ref,
                o_ref, p_ref, qr_ref):
    x = x_ref[...]
    xw = jnp.dot(x, w1_ref[...], preferred_element_type=jnp.float32)
    col = lax.broadcasted_iota(jnp.int32, (N, TW - H), 1)
    pad = jnp.where(col == 0, 1.0, 0.0).astype(jnp.float32)
    o_ref[...] = jnp.concatenate([xw, pad], axis=1)
    xm = x * (lax.broadcasted_iota(jnp.int32, (1, D), 1) > 0
              ).astype(jnp.float32)
    p = jnp.dot(jnp.dot(xm, wdecpt_ref[...],
                        preferred_element_type=jnp.float32),
                wmut_ref[...], preferred_element_type=jnp.float32)
    p_ref[...] = p
    q = jnp.sum(xm * bdecp_ref[...], axis=1, keepdims=True)
    r = jnp.sum(xm * xm, axis=1, keepdims=True)
    zcol = jnp.zeros((N, 6), jnp.float32)
    qr_ref[...] = jnp.concatenate([q, r, zcol], axis=1)


def _sc_body(table_hbm, srcm_hbm, dstm_hbm, zeros_hbm, out_hbm,
             srcv, dstv, rows, acc, gsem, ssem):
    cid = lax.axis_index("c")
    sid = lax.axis_index("s")
    wid = sid * NC + cid

    pltpu.sync_copy(zeros_hbm, acc.at[pl.ds(sid * ZR, ZR)])
    pltpu.sync_copy(srcm_hbm.at[pl.ds(wid * NCHUNK, NCHUNK)], srcv)
    pltpu.sync_copy(dstm_hbm.at[pl.ds(wid * NCHUNK, NCHUNK)], dstv)
    plsc.subcore_barrier()

    def start_gather(j, b):
        pltpu.async_copy(table_hbm.at[srcv.at[j]], rows.at[b], gsem.at[b])

    def wait_gather(j, b):
        pltpu.make_async_copy(table_hbm.at[srcv.at[j]], rows.at[b],
                              gsem.at[b]).wait()

    def start_scatter(j, b):
        pltpu.async_copy(rows.at[b], acc.at[dstv.at[j]], ssem.at[b], add=True)

    def wait_scatter(j, b):
        pltpu.make_async_copy(rows.at[b], acc.at[dstv.at[j]],
                              ssem.at[b]).wait()

    for b in range(NBUF):
        start_gather(b, b)

    def outer(g, carry):
        for b in range(NBUF):
            j = g * NBUF + b
            wait_gather(j, b)
            start_scatter(j, b)
        for b in range(NBUF):
            j = g * NBUF + b
            wait_scatter(j, b)

            @pl.when(g < NOUTER - 1)
            def _():
                start_gather(j + NBUF, b)
        return carry

    lax.fori_loop(0, NOUTER, outer, 0)
    plsc.subcore_barrier()
    pltpu.sync_copy(acc.at[pl.ds(sid * ZR, ZR)],
                    out_hbm.at[pl.ds(cid * N + sid * ZR, ZR)])


@functools.lru_cache(maxsize=1)
def _make_sc_call():
    return pl.kernel(
        _sc_body,
        out_type=jax.ShapeDtypeStruct((NC * N, TW), jnp.float32),
        mesh=plsc.VectorSubcoreMesh(core_axis_name="c", subcore_axis_name="s",
                                    num_cores=NC, num_subcores=NS),
        scratch_types=[
            pltpu.VMEM((NCHUNK, CH), jnp.int32),
            pltpu.VMEM((NCHUNK, CH), jnp.int32),
            pltpu.VMEM((NBUF, CH, TW), jnp.float32),
            pltpu.VMEM_SHARED((N, TW), jnp.float32),
            pltpu.SemaphoreType.DMA((NBUF,)),
            pltpu.SemaphoreType.DMA((NBUF,)),
        ],
        compiler_params=pltpu.CompilerParams(use_tc_tiling_on_sc=False),
    )


def _final_body(table_ref, acc_ref, p_ref, qr_ref, b1_ref, wmu_ref,
                wdecp_ref, wdecpt_ref, wmut_ref, bdecp_ref, batch_ref,
                o_ref):
    accs = acc_ref[0:N, :] + acc_ref[N:2 * N, :]
    seg = accs[:, 0:H]
    deg = accs[:, H:H + 1]
    xw = table_ref[:, 0:H]
    h = jnp.maximum(xw + seg / jnp.maximum(deg, 1.0) + b1_ref[...], 0.0)
    wc = jnp.dot(wmu_ref[...], wdecp_ref[...],
                 preferred_element_type=jnp.float32)
    wct = jnp.dot(wdecpt_ref[...], wmut_ref[...],
                  preferred_element_type=jnp.float32)
    a_mat = jnp.dot(wc, wct, preferred_element_type=jnp.float32)
    u_row = jnp.dot(bdecp_ref[...], wct,
                    preferred_element_type=jnp.float32)
    s = jnp.sum(bdecp_ref[...] * bdecp_ref[...])
    ha = jnp.dot(h, a_mat, preferred_element_type=jnp.float32)
    t1 = jnp.sum(ha * h, axis=1, keepdims=True)
    t2 = jnp.sum(h * u_row, axis=1, keepdims=True)
    t3 = jnp.sum(h * p_ref[...], axis=1, keepdims=True)
    q = qr_ref[:, 0:1]
    r = qr_ref[:, 1:2]
    ne = (t1 + 2.0 * t2 + s - 2.0 * (t3 + q) + r) * (1.0 / (D - 1))
    ne2 = jnp.concatenate([ne, jnp.ones((N, 1), jnp.float32)], axis=1)
    oh_t = (batch_ref[...] == lax.broadcasted_iota(jnp.int32, (G, N), 0)
            ).astype(jnp.float32)
    res = jnp.dot(oh_t, ne2, preferred_element_type=jnp.float32)
    ge = res[:, 0:1] / jnp.maximum(res[:, 1:2], 1.0)
    o_ref[...] = jnp.concatenate([-ge, ge], axis=1)


def kernel(x, edge_index, batch, W1, b1, Wmu, Wdec, bdec):
    wdecp = jnp.pad(Wdec, ((0, 0), (1, 0)))
    bdecp = jnp.pad(bdec, (1, 0)).reshape(1, D)
    wdecpt = wdecp.T
    wmut = Wmu.T

    table, p, qr = pl.pallas_call(
        _table_body,
        out_shape=[
            jax.ShapeDtypeStruct((N, TW), jnp.float32),
            jax.ShapeDtypeStruct((N, H), jnp.float32),
            jax.ShapeDtypeStruct((N, 8), jnp.float32),
        ],
    )(x, W1, wdecpt, wmut, bdecp)

    srcm = edge_index[0].reshape(NW * NCHUNK, CH)
    dstm = edge_index[1].reshape(NW * NCHUNK, CH)
    zeros_blk = jnp.zeros((ZR, TW), jnp.float32)
    acc = _make_sc_call()(table, srcm, dstm, zeros_blk)

    return pl.pallas_call(
        _final_body,
        out_shape=jax.ShapeDtypeStruct((G, 2), jnp.float32),
    )(table, acc, p, qr, b1.reshape(1, H), Wmu, wdecp, wdecpt, wmut,
      bdecp, batch.reshape(1, N))
